# baseline (speedup 1.0000x reference)
import functools

import jax
import jax.numpy as jnp
from jax import lax
from jax.experimental import pallas as pl
from jax.experimental.pallas import tpu as pltpu
from jax.experimental.pallas import tpu_sc as plsc

EMB = 64
B = 4096
L = 50

VOCAB_PAD = 100096
HALF = VOCAB_PAD // 2

NC, NS = 2, 16
NW = NC * NS
RPW = B // NW
CB = 2
G = RPW // CB
LANES = 16
EC = EMB // LANES

_RJ = 391


def _repack_body(a_ref, b_ref, o_ref):
    o_ref[...] = jnp.concatenate([a_ref[...].T, b_ref[...].T], axis=1)


_repack = pl.pallas_call(
    _repack_body,
    grid=(_RJ,),
    in_specs=[
        pl.BlockSpec((EMB, 128), lambda j: (0, j)),
        pl.BlockSpec((EMB, 128), lambda j: (0, j + _RJ)),
    ],
    out_specs=pl.BlockSpec((128, 128), lambda j: (j, 0)),
    out_shape=jax.ShapeDtypeStruct((HALF, 128), jnp.float32),
)

_mesh = plsc.VectorSubcoreMesh(
    core_axis_name="c", subcore_axis_name="s", num_cores=NC, num_subcores=NS
)


@functools.partial(
    pl.kernel,
    out_type=jax.ShapeDtypeStruct((NW, RPW, EMB), jnp.float32),
    mesh=_mesh,
    compiler_params=pltpu.CompilerParams(use_tc_tiling_on_sc=False),
    scratch_types=[
        pltpu.VMEM((G, CB * L), jnp.int32),
        pltpu.VMEM((CB * L, EMB), jnp.float32),
        pltpu.VMEM((CB * L, EMB), jnp.float32),
        pltpu.VMEM((RPW, EMB), jnp.float32),
        pltpu.SemaphoreType.DMA,
        pltpu.SemaphoreType.DMA,
    ],
)
def _encode(x_hbm, table_hbm, out_hbm, idx_v, rows0, rows1, out_v, sem0, sem1):
    wid = lax.axis_index("s") * NC + lax.axis_index("c")
    pltpu.sync_copy(x_hbm.at[wid], idx_v)

    scale = jnp.float32(1.0 / L)
    bufs = (rows0, rows1)
    sems = (sem0, sem1)

    def compute(g, buf):
        for r in range(CB):
            accs = [buf[r * L, pl.ds(c * LANES, LANES)] for c in range(EC)]
            for l in range(1, L):
                for c in range(EC):
                    accs[c] = accs[c] + buf[r * L + l, pl.ds(c * LANES, LANES)]
            for c in range(EC):
                out_v[g * CB + r, pl.ds(c * LANES, LANES)] = accs[c] * scale

    pltpu.async_copy(table_hbm.at[idx_v.at[0]], bufs[0], sems[0])

    @pl.loop(0, G, step=2)
    def _pair(g):
        for b in range(2):
            gg = g + b

            @pl.when(gg + 1 < G)
            def _():
                pltpu.async_copy(
                    table_hbm.at[idx_v.at[gg + 1]], bufs[1 - b], sems[1 - b]
                )

            pltpu.make_async_copy(
                table_hbm.at[idx_v.at[gg]], bufs[b], sems[b]
            ).wait()
            compute(gg, bufs[b])

    pltpu.sync_copy(out_v, out_hbm.at[wid])


def kernel(x, table):
    tt = table.T
    packed = _repack(tt, tt)
    table_lin = packed.reshape(VOCAB_PAD, EMB)
    x2 = jnp.where(x < HALF, x * 2, (x - HALF) * 2 + 1)
    xr = x2.reshape(NW, G, CB * L)
    out = _encode(xr, table_lin)
    return out.reshape(B, EMB)


# speedup vs baseline: 1.8220x; 1.8220x over previous
import functools

import jax
import jax.numpy as jnp
from jax import lax
from jax.experimental import pallas as pl
from jax.experimental.pallas import tpu as pltpu
from jax.experimental.pallas import tpu_sc as plsc

EMB = 64
B = 4096
L = 50

VOCAB_PAD = 100096
HALF = VOCAB_PAD // 2

NC, NS = 2, 16
NW = NC * NS
RPW = B // NW
CB = 2
G = RPW // CB
LANES = 16
EC = EMB // LANES

_RJ = 391


def _pack_table(table):
    tt = table.T
    ttp = jnp.pad(tt, ((0, 0), (0, VOCAB_PAD - tt.shape[1])))
    return ttp.reshape(EMB, HALF, 2).transpose(1, 2, 0).reshape(VOCAB_PAD, EMB)

_mesh = plsc.VectorSubcoreMesh(
    core_axis_name="c", subcore_axis_name="s", num_cores=NC, num_subcores=NS
)


@functools.partial(
    pl.kernel,
    out_type=jax.ShapeDtypeStruct((NW, RPW, EMB), jnp.float32),
    mesh=_mesh,
    compiler_params=pltpu.CompilerParams(use_tc_tiling_on_sc=False),
    scratch_types=[
        pltpu.VMEM((G, CB * L), jnp.int32),
        pltpu.VMEM((CB * L, EMB), jnp.float32),
        pltpu.VMEM((CB * L, EMB), jnp.float32),
        pltpu.VMEM((RPW, EMB), jnp.float32),
        pltpu.SemaphoreType.DMA,
        pltpu.SemaphoreType.DMA,
    ],
)
def _encode(x_hbm, table_hbm, out_hbm, idx_v, rows0, rows1, out_v, sem0, sem1):
    wid = lax.axis_index("s") * NC + lax.axis_index("c")
    pltpu.sync_copy(x_hbm.at[wid], idx_v)

    scale = jnp.float32(1.0 / L)
    bufs = (rows0, rows1)
    sems = (sem0, sem1)

    def compute(g, buf):
        for r in range(CB):
            accs = [buf[r * L, pl.ds(c * LANES, LANES)] for c in range(EC)]
            for l in range(1, L):
                for c in range(EC):
                    accs[c] = accs[c] + buf[r * L + l, pl.ds(c * LANES, LANES)]
            for c in range(EC):
                out_v[g * CB + r, pl.ds(c * LANES, LANES)] = accs[c] * scale

    pltpu.async_copy(table_hbm.at[idx_v.at[0]], bufs[0], sems[0])

    @pl.loop(0, G, step=2)
    def _pair(g):
        for b in range(2):
            gg = g + b

            @pl.when(gg + 1 < G)
            def _():
                pltpu.async_copy(
                    table_hbm.at[idx_v.at[gg + 1]], bufs[1 - b], sems[1 - b]
                )

            pltpu.make_async_copy(
                table_hbm.at[idx_v.at[gg]], bufs[b], sems[b]
            ).wait()
            compute(gg, bufs[b])

    pltpu.sync_copy(out_v, out_hbm.at[wid])


def kernel(x, table):
    table_lin = _pack_table(table)
    xr = x.reshape(NW, G, CB * L)
    out = _encode(xr, table_lin)
    return out.reshape(B, EMB)


# speedup vs baseline: 2.9394x; 1.6133x over previous
import functools

import jax
import jax.numpy as jnp
from jax import lax
from jax.experimental import pallas as pl
from jax.experimental.pallas import tpu as pltpu
from jax.experimental.pallas import tpu_sc as plsc

EMB = 64
B = 4096
L = 50

VOCAB_PAD = 100096
HALF = VOCAB_PAD // 2

NC, NS = 2, 16
NW = NC * NS
RPW = B // NW
CB = 2
G = RPW // CB
LANES = 16
EC = EMB // LANES

_RJ = 391


_RW = 2176


def _repack_body(a_ref, b_ref, o_ref):
    r = lax.broadcasted_iota(jnp.int32, (EMB, 2 * EMB), 0)
    c = lax.broadcasted_iota(jnp.int32, (EMB, 2 * EMB), 1)
    e1 = (r == c).astype(jnp.float32)
    e2 = (r + EMB == c).astype(jnp.float32)
    dn = (((0,), (0,)), ((), ()))
    o_ref[...] = lax.dot_general(
        a_ref[...], e1, dn, preferred_element_type=jnp.float32
    ) + lax.dot_general(b_ref[...], e2, dn, preferred_element_type=jnp.float32)


_repack = pl.pallas_call(
    _repack_body,
    grid=(HALF // _RW,),
    in_specs=[
        pl.BlockSpec((EMB, _RW), lambda j: (0, j)),
        pl.BlockSpec((EMB, _RW), lambda j: (0, j + HALF // _RW)),
    ],
    out_specs=pl.BlockSpec((_RW, 2 * EMB), lambda j: (j, 0)),
    out_shape=jax.ShapeDtypeStruct((HALF, 2 * EMB), jnp.float32),
)


def _pack_table(table):
    tt = table.T
    return _repack(tt, tt).reshape(VOCAB_PAD, EMB)

_mesh = plsc.VectorSubcoreMesh(
    core_axis_name="c", subcore_axis_name="s", num_cores=NC, num_subcores=NS
)


@functools.partial(
    pl.kernel,
    out_type=jax.ShapeDtypeStruct((NW, RPW, EMB), jnp.float32),
    mesh=_mesh,
    compiler_params=pltpu.CompilerParams(use_tc_tiling_on_sc=False),
    scratch_types=[
        pltpu.VMEM((G, CB * L), jnp.int32),
        pltpu.VMEM((CB * L, EMB), jnp.float32),
        pltpu.VMEM((CB * L, EMB), jnp.float32),
        pltpu.VMEM((RPW, EMB), jnp.float32),
        pltpu.SemaphoreType.DMA,
        pltpu.SemaphoreType.DMA,
    ],
)
def _encode(x_hbm, table_hbm, out_hbm, idx_v, rows0, rows1, out_v, sem0, sem1):
    wid = lax.axis_index("s") * NC + lax.axis_index("c")
    pltpu.sync_copy(x_hbm.at[wid], idx_v)

    scale = jnp.float32(1.0 / L)
    bufs = (rows0, rows1)
    sems = (sem0, sem1)

    def compute(g, buf):
        for r in range(CB):
            accs = [buf[r * L, pl.ds(c * LANES, LANES)] for c in range(EC)]
            for l in range(1, L):
                for c in range(EC):
                    accs[c] = accs[c] + buf[r * L + l, pl.ds(c * LANES, LANES)]
            for c in range(EC):
                out_v[g * CB + r, pl.ds(c * LANES, LANES)] = accs[c] * scale

    pltpu.async_copy(table_hbm.at[idx_v.at[0]], bufs[0], sems[0])

    @pl.loop(0, G, step=2)
    def _pair(g):
        for b in range(2):
            gg = g + b

            @pl.when(gg + 1 < G)
            def _():
                pltpu.async_copy(
                    table_hbm.at[idx_v.at[gg + 1]], bufs[1 - b], sems[1 - b]
                )

            pltpu.make_async_copy(
                table_hbm.at[idx_v.at[gg]], bufs[b], sems[b]
            ).wait()
            compute(gg, bufs[b])

    pltpu.sync_copy(out_v, out_hbm.at[wid])


def kernel(x, table):
    table_lin = _pack_table(table)
    x2 = jnp.where(x < HALF, x * 2, (x - HALF) * 2 + 1)
    xr = x2.reshape(NW, G, CB * L)
    out = _encode(xr, table_lin)
    return out.reshape(B, EMB)
